# baseline (speedup 1.0000x reference)
import functools

import jax
import jax.numpy as jnp
from jax import lax
from jax.experimental import pallas as pl
from jax.experimental.pallas import tpu as pltpu
from jax.experimental.pallas import tpu_sc as plsc

_NC = 2
_NS = 16
_C = 128
_B = 1000


_NB = 4


def _seg_sum_sc(y, src, dst, zrow, npad):
    N, F = y.shape
    num_chunks = src.shape[0] // _C
    rpt = npad // _NS
    cpt = num_chunks // (_NC * _NS)
    mesh = plsc.VectorSubcoreMesh(core_axis_name="c", subcore_axis_name="s")

    @functools.partial(
        pl.kernel,
        out_type=jax.ShapeDtypeStruct((_NC * npad, F), jnp.float32),
        mesh=mesh,
        scratch_types=[
            pltpu.VMEM((_C,), jnp.int32),
            pltpu.VMEM((_C,), jnp.int32),
            pltpu.VMEM((_C,), jnp.int32),
            pltpu.VMEM((_C,), jnp.int32),
            pltpu.VMEM((_C, F), jnp.float32),
            pltpu.VMEM((_C, F), jnp.float32),
            pltpu.VMEM_SHARED((npad, F), jnp.float32),
            pltpu.SemaphoreType.DMA,
            pltpu.SemaphoreType.DMA,
        ],
        compiler_params=pltpu.CompilerParams(use_tc_tiling_on_sc=False),
    )
    def agg(y_hbm, src_hbm, dst_hbm, z_hbm, out_hbm, src_a, src_b,
            dst_a, dst_b, rows_a, rows_b, acc_sh, sem_a, sem_b):
        cid = lax.axis_index("c")
        sid = lax.axis_index("s")
        wid = sid * _NC + cid
        row0 = sid * rpt
        pltpu.sync_copy(z_hbm, acc_sh.at[pl.ds(row0, rpt)])
        plsc.subcore_barrier()
        e0 = wid * cpt * _C

        def body(i, carry):
            base = e0 + 2 * i * _C
            pltpu.sync_copy(src_hbm.at[pl.ds(base, _C)], src_a)
            pltpu.sync_copy(dst_hbm.at[pl.ds(base, _C)], dst_a)
            ha = pltpu.async_copy(y_hbm.at[src_a], rows_a, sem_a)
            pltpu.sync_copy(src_hbm.at[pl.ds(base + _C, _C)], src_b)
            pltpu.sync_copy(dst_hbm.at[pl.ds(base + _C, _C)], dst_b)
            hb = pltpu.async_copy(y_hbm.at[src_b], rows_b, sem_b)
            ha.wait()
            pltpu.sync_copy(rows_a, acc_sh.at[dst_a], add=True)
            hb.wait()
            pltpu.sync_copy(rows_b, acc_sh.at[dst_b], add=True)
            return carry

        lax.fori_loop(0, cpt // 2, body, 0)
        plsc.subcore_barrier()
        pltpu.sync_copy(acc_sh.at[pl.ds(row0, rpt)],
                        out_hbm.at[pl.ds(cid * npad + row0, rpt)])

    return agg(y, src, dst, zrow)


def _mm_tc(x, W):
    N, D = x.shape
    H = W.shape[1]

    def body(x_ref, w_ref, o_ref):
        o_ref[...] = jnp.dot(x_ref[...], w_ref[...],
                             preferred_element_type=jnp.float32)

    return pl.pallas_call(
        body,
        grid=(N // _B,),
        in_specs=[
            pl.BlockSpec((_B, D), lambda i: (i, 0)),
            pl.BlockSpec((D, H), lambda i: (0, 0)),
        ],
        out_specs=pl.BlockSpec((_B, H), lambda i: (i, 0)),
        out_shape=jax.ShapeDtypeStruct((N, H), jnp.float32),
    )(x, W)


def _mid_tc(opeps, y, p0, p1, ba, Wb, bb, Wnext):
    N, H = y.shape
    H2 = Wnext.shape[1]

    def body(e_ref, y_ref, p0_ref, p1_ref, ba_ref, wb_ref, bb_ref, wn_ref,
             o_ref):
        e = e_ref[0]
        t = jnp.maximum(e * y_ref[...] + p0_ref[...] + p1_ref[...]
                        + ba_ref[...], 0.0)
        h = jnp.dot(t, wb_ref[...], preferred_element_type=jnp.float32)
        h = jnp.maximum(h + bb_ref[...], 0.0)
        o_ref[...] = jnp.dot(h, wn_ref[...],
                             preferred_element_type=jnp.float32)

    return pl.pallas_call(
        body,
        grid=(N // _B,),
        in_specs=[
            pl.BlockSpec(memory_space=pltpu.SMEM),
            pl.BlockSpec((_B, H), lambda i: (i, 0)),
            pl.BlockSpec((_B, H), lambda i: (i, 0)),
            pl.BlockSpec((_B, H), lambda i: (i, 0)),
            pl.BlockSpec((1, H), lambda i: (0, 0)),
            pl.BlockSpec((H, H), lambda i: (0, 0)),
            pl.BlockSpec((1, H), lambda i: (0, 0)),
            pl.BlockSpec((H, H2), lambda i: (0, 0)),
        ],
        out_specs=pl.BlockSpec((_B, H2), lambda i: (i, 0)),
        out_shape=jax.ShapeDtypeStruct((N, H2), jnp.float32),
    )(opeps, y, p0, p1, ba, Wb, bb, Wnext)


def _head_tc(opeps, z, q0, q1, ba, Wb, bb, Wout, bout):
    N, H = z.shape
    O = Wout.shape[1]

    def body(e_ref, z_ref, q0_ref, q1_ref, ba_ref, wb_ref, bb_ref, wo_ref,
             bo_ref, o_ref):
        e = e_ref[0]
        t = jnp.maximum(e * z_ref[...] + q0_ref[...] + q1_ref[...]
                        + ba_ref[...], 0.0)
        h = jnp.dot(t, wb_ref[...], preferred_element_type=jnp.float32)
        h = jnp.maximum(h + bb_ref[...], 0.0)
        logits = jnp.dot(h, wo_ref[...],
                         preferred_element_type=jnp.float32) + bo_ref[...]
        m = jnp.max(logits, axis=-1, keepdims=True)
        s = logits - m
        lse = jnp.log(jnp.sum(jnp.exp(s), axis=-1, keepdims=True))
        o_ref[...] = s - lse

    return pl.pallas_call(
        body,
        grid=(N // _B,),
        in_specs=[
            pl.BlockSpec(memory_space=pltpu.SMEM),
            pl.BlockSpec((_B, H), lambda i: (i, 0)),
            pl.BlockSpec((_B, H), lambda i: (i, 0)),
            pl.BlockSpec((_B, H), lambda i: (i, 0)),
            pl.BlockSpec((1, H), lambda i: (0, 0)),
            pl.BlockSpec((H, H), lambda i: (0, 0)),
            pl.BlockSpec((1, H), lambda i: (0, 0)),
            pl.BlockSpec((H, O), lambda i: (0, 0)),
            pl.BlockSpec((1, O), lambda i: (0, 0)),
        ],
        out_specs=pl.BlockSpec((_B, O), lambda i: (i, 0)),
        out_shape=jax.ShapeDtypeStruct((N, O), jnp.float32),
    )(opeps, z, q0, q1, ba, Wb, bb, Wout, bout)


def kernel(x, edge_index, eps0, W0a, b0a, W0b, b0b, g0, be0,
           eps1, W1a, b1a, W1b, b1b, g1, be1, Wout, bout):
    N = x.shape[0]
    E = edge_index.shape[1]
    npad = -(-N // (8 * _NS)) * (8 * _NS)
    if npad - N < 1:
        npad += 8 * _NS
    nw = _NC * _NS
    cpt = -(-(-(-E // _C)) // (nw * 2 * _NB)) * (2 * _NB)
    epad = nw * cpt * _C - E
    src = jnp.concatenate([edge_index[0], jnp.zeros((epad,), jnp.int32)])
    dst = jnp.concatenate(
        [edge_index[1],
         N + (jnp.arange(epad, dtype=jnp.int32) % (npad - N))])
    zrow = jnp.zeros((npad // _NS, W0a.shape[1]), jnp.float32)

    bn = 1.0 / jnp.sqrt(jnp.float32(1.0 + 1e-5))
    s0 = g0 * bn
    W0bf = W0b * s0[None, :]
    b0bf = (b0b * s0 + be0)[None, :]
    s1 = g1 * bn
    W1bf = W1b * s1[None, :]
    b1bf = (b1b * s1 + be1)[None, :]

    ope0 = jnp.reshape(1.0 + eps0, (1,))
    ope1 = jnp.reshape(1.0 + eps1, (1,))

    y0 = _mm_tc(x, W0a)
    parts0 = _seg_sum_sc(y0, src, dst, zrow, npad)
    p0a = parts0[:N]
    p0b = parts0[npad:npad + N]
    z = _mid_tc(ope0, y0, p0a, p0b, b0a[None, :], W0bf, b0bf, W1a)

    parts1 = _seg_sum_sc(z, src, dst, zrow, npad)
    q0 = parts1[:N]
    q1 = parts1[npad:npad + N]
    return _head_tc(ope1, z, q0, q1, b1a[None, :], W1bf, b1bf,
                    Wout, bout[None, :])


# speedup vs baseline: 2.9383x; 2.9383x over previous
import functools

import jax
import jax.numpy as jnp
from jax import lax
from jax.experimental import pallas as pl
from jax.experimental.pallas import tpu as pltpu
from jax.experimental.pallas import tpu_sc as plsc

_NC = 2
_NS = 16
_C = 512
_BP = 1000
_NB = 2


def _seg_sum_sc(y, src, dst, zrow, npad):
    N, F = y.shape
    num_chunks = src.shape[0] // _C
    rpt = npad // _NS
    cpt = num_chunks // (_NC * _NS)
    mesh = plsc.VectorSubcoreMesh(core_axis_name="c", subcore_axis_name="s")

    @functools.partial(
        pl.kernel,
        out_type=jax.ShapeDtypeStruct((_NC * npad, F), jnp.float32),
        mesh=mesh,
        scratch_types=[
            [pltpu.VMEM((_C,), jnp.int32) for _ in range(_NB)],
            [pltpu.VMEM((_C,), jnp.int32) for _ in range(_NB)],
            [pltpu.VMEM((_C, F), jnp.float32) for _ in range(_NB)],
            pltpu.VMEM_SHARED((npad, F), jnp.float32),
            [pltpu.SemaphoreType.DMA for _ in range(_NB)],
        ],
        compiler_params=pltpu.CompilerParams(use_tc_tiling_on_sc=False),
    )
    def agg(y_hbm, src_hbm, dst_hbm, z_hbm, out_hbm, srcs, dsts, rows,
            acc_sh, sems):
        cid = lax.axis_index("c")
        sid = lax.axis_index("s")
        wid = sid * _NC + cid
        row0 = sid * rpt
        pltpu.sync_copy(z_hbm, acc_sh.at[pl.ds(row0, rpt)])
        plsc.subcore_barrier()
        e0 = wid * cpt * _C

        def body(i, carry):
            base = e0 + _NB * i * _C
            hs = []
            for b in range(_NB):
                pltpu.sync_copy(src_hbm.at[pl.ds(base + b * _C, _C)], srcs[b])
                pltpu.sync_copy(dst_hbm.at[pl.ds(base + b * _C, _C)], dsts[b])
                hs.append(pltpu.async_copy(y_hbm.at[srcs[b]], rows[b],
                                           sems[b]))
            for b in range(_NB):
                hs[b].wait()
                pltpu.sync_copy(rows[b], acc_sh.at[dsts[b]], add=True)
            return carry

        lax.fori_loop(0, cpt // _NB, body, 0)
        plsc.subcore_barrier()
        pltpu.sync_copy(acc_sh.at[pl.ds(row0, rpt)],
                        out_hbm.at[pl.ds(cid * npad + row0, rpt)])

    return agg(y, src, dst, zrow)


def _mm_tc(x, W):
    Np, D = x.shape
    H = W.shape[1]

    def body(x_ref, w_ref, o_ref):
        o_ref[...] = jnp.dot(x_ref[...], w_ref[...],
                             preferred_element_type=jnp.float32)

    return pl.pallas_call(
        body,
        grid=(Np // _BP,),
        in_specs=[
            pl.BlockSpec((_BP, D), lambda i: (i, 0)),
            pl.BlockSpec((D, H), lambda i: (0, 0)),
        ],
        out_specs=pl.BlockSpec((_BP, H), lambda i: (i, 0)),
        out_shape=jax.ShapeDtypeStruct((Np, H), jnp.float32),
    )(x, W)


def _mid_tc(opeps, y, p0, p1, ba, Wb, bb, Wnext):
    Np, H = y.shape
    H2 = Wnext.shape[1]

    def body(e_ref, y_ref, p0_ref, p1_ref, ba_ref, wb_ref, bb_ref, wn_ref,
             o_ref):
        e = e_ref[0]
        t = jnp.maximum(e * y_ref[...] + p0_ref[...] + p1_ref[...]
                        + ba_ref[...], 0.0)
        h = jnp.dot(t, wb_ref[...], preferred_element_type=jnp.float32)
        h = jnp.maximum(h + bb_ref[...], 0.0)
        o_ref[...] = jnp.dot(h, wn_ref[...],
                             preferred_element_type=jnp.float32)

    return pl.pallas_call(
        body,
        grid=(Np // _BP,),
        in_specs=[
            pl.BlockSpec(memory_space=pltpu.SMEM),
            pl.BlockSpec((_BP, H), lambda i: (i, 0)),
            pl.BlockSpec((_BP, H), lambda i: (i, 0)),
            pl.BlockSpec((_BP, H), lambda i: (i, 0)),
            pl.BlockSpec((1, H), lambda i: (0, 0)),
            pl.BlockSpec((H, H), lambda i: (0, 0)),
            pl.BlockSpec((1, H), lambda i: (0, 0)),
            pl.BlockSpec((H, H2), lambda i: (0, 0)),
        ],
        out_specs=pl.BlockSpec((_BP, H2), lambda i: (i, 0)),
        out_shape=jax.ShapeDtypeStruct((Np, H2), jnp.float32),
    )(opeps, y, p0, p1, ba, Wb, bb, Wnext)


def _head_tc(opeps, z, q0, q1, ba, Wb, bb, Wout, bout, O):
    Np, H = z.shape
    O2 = Wout.shape[1]

    def body(e_ref, z_ref, q0_ref, q1_ref, ba_ref, wb_ref, bb_ref, wo_ref,
             bo_ref, o_ref):
        e = e_ref[0]
        t = jnp.maximum(e * z_ref[...] + q0_ref[...] + q1_ref[...]
                        + ba_ref[...], 0.0)
        h = jnp.dot(t, wb_ref[...], preferred_element_type=jnp.float32)
        h = jnp.maximum(h + bb_ref[...], 0.0)
        lg = jnp.dot(h, wo_ref[...],
                     preferred_element_type=jnp.float32) + bo_ref[...]
        outs = []
        for k in range(2):
            l = lg[:, k * O:(k + 1) * O]
            s = l - jnp.max(l, axis=-1, keepdims=True)
            outs.append(s - jnp.log(jnp.sum(jnp.exp(s), axis=-1,
                                            keepdims=True)))
        o_ref[...] = jnp.concatenate(outs, axis=-1)

    return pl.pallas_call(
        body,
        grid=(Np // _BP,),
        in_specs=[
            pl.BlockSpec(memory_space=pltpu.SMEM),
            pl.BlockSpec((_BP, H), lambda i: (i, 0)),
            pl.BlockSpec((_BP, H), lambda i: (i, 0)),
            pl.BlockSpec((_BP, H), lambda i: (i, 0)),
            pl.BlockSpec((1, H), lambda i: (0, 0)),
            pl.BlockSpec((H, H), lambda i: (0, 0)),
            pl.BlockSpec((1, H), lambda i: (0, 0)),
            pl.BlockSpec((H, O2), lambda i: (0, 0)),
            pl.BlockSpec((1, O2), lambda i: (0, 0)),
        ],
        out_specs=pl.BlockSpec((_BP, O2), lambda i: (i, 0)),
        out_shape=jax.ShapeDtypeStruct((Np, O2), jnp.float32),
    )(opeps, z, q0, q1, ba, Wb, bb, Wout, bout)


def _bd(W):
    Z = jnp.zeros(W.shape, W.dtype)
    return jnp.block([[W, Z], [Z, W]])


def _b2(b):
    return jnp.concatenate([b, b])[None, :]


def kernel(x, edge_index, eps0, W0a, b0a, W0b, b0b, g0, be0,
           eps1, W1a, b1a, W1b, b1b, g1, be1, Wout, bout):
    N, D = x.shape
    H = W0a.shape[1]
    O = Wout.shape[1]
    E = edge_index.shape[1]
    Np = N // 2
    npad = -(-N // (8 * _NS)) * (8 * _NS)
    if npad - N < 1:
        npad += 8 * _NS
    nw = _NC * _NS
    cpt = -(-(-(-E // _C)) // (nw * _NB)) * _NB
    epad = nw * cpt * _C - E
    src = jnp.concatenate(
        [edge_index[0], jnp.arange(epad, dtype=jnp.int32) % N])
    dst = jnp.concatenate(
        [edge_index[1],
         N + (jnp.arange(epad, dtype=jnp.int32) % (npad - N))])
    zrow = jnp.zeros((npad // _NS, H), jnp.float32)

    bn = 1.0 / jnp.sqrt(jnp.float32(1.0 + 1e-5))
    s0 = g0 * bn
    W0bf = W0b * s0[None, :]
    b0bf = b0b * s0 + be0
    s1 = g1 * bn
    W1bf = W1b * s1[None, :]
    b1bf = b1b * s1 + be1

    ope0 = jnp.reshape(1.0 + eps0, (1,))
    ope1 = jnp.reshape(1.0 + eps1, (1,))

    x_pair = x.reshape(Np, 2 * D)

    y0p = _mm_tc(x_pair, _bd(W0a))
    parts0 = _seg_sum_sc(y0p.reshape(N, H), src, dst, zrow, npad)
    pp = parts0.reshape(2, npad // 2, 2 * H)
    zp = _mid_tc(ope0, y0p, pp[0], pp[1], _b2(b0a), _bd(W0bf), _b2(b0bf),
                 _bd(W1a))

    parts1 = _seg_sum_sc(zp.reshape(N, H), src, dst, zrow, npad)
    qq = parts1.reshape(2, npad // 2, 2 * H)
    lgp = _head_tc(ope1, zp, qq[0], qq[1], _b2(b1a), _bd(W1bf), _b2(b1bf),
                   _bd(Wout), _b2(bout), O)
    return lgp.reshape(N, O)
